# baseline (speedup 1.0000x reference)
import jax
import jax.numpy as jnp
from jax import lax
from jax.experimental import pallas as pl
from jax.experimental.pallas import tpu as pltpu
from jax.experimental.pallas import tpu_sc as plsc

N = 100000
NC, NS = 2, 16
CHUNK = 512
D = 2
SUP = D * CHUNK
R = 100352
ZROWS = R // NS
BN = 2000

_mesh = plsc.VectorSubcoreMesh(
    core_axis_name="c", subcore_axis_name="s", num_cores=NC, num_subcores=NS)


def _edge_loop(edges, tab, acc, idxb, rows, semi, semg, sems,
               row_base, n_rows):
  @pl.loop(0, n_rows // D)
  def _outer(i):
    g0 = row_base + i * D
    il = [pltpu.async_copy(edges.at[g0 + b], idxb.at[b], semi)
          for b in range(D)]
    gl = []
    for b in range(D):
      il[b].wait()
      gl.append(pltpu.async_copy(tab.at[idxb.at[b, 0]], rows.at[b], semg))
    sl = []
    for b in range(D):
      gl[b].wait()
      sl.append(pltpu.async_copy(rows.at[b], acc.at[idxb.at[b, 1]],
                                 sems, add=True))
    for b in range(D):
      sl[b].wait()


def _make_scatter(F, total_rows):
  def body(edges, ta, tb, zer, out, idxb, rows, acc, semi, semg, sems):
    c = lax.axis_index("c")
    s = lax.axis_index("s")
    sl = pl.ds(s * ZROWS, ZROWS)
    pltpu.sync_copy(zer, acc.at[sl])
    plsc.subcore_barrier()
    rpt = total_rows // NS
    base = s * rpt
    pl.when(c == 0)(lambda: _edge_loop(
        edges, ta, acc, idxb, rows, semi, semg, sems, base, rpt))
    pl.when(c == 1)(lambda: _edge_loop(
        edges, tb, acc, idxb, rows, semi, semg, sems, base, rpt))
    plsc.subcore_barrier()
    pl.when(c == 0)(lambda: pltpu.sync_copy(acc.at[sl], out.at[0, sl]))
    pl.when(c == 1)(lambda: pltpu.sync_copy(acc.at[sl], out.at[1, sl]))

  return pl.kernel(
      body,
      out_type=jax.ShapeDtypeStruct((NC, R, F), jnp.float32),
      mesh=_mesh,
      compiler_params=pltpu.CompilerParams(use_tc_tiling_on_sc=False),
      scratch_types=[
          pltpu.VMEM((D, 2, CHUNK), jnp.int32),
          pltpu.VMEM((D, CHUNK, F), jnp.float32),
          pltpu.VMEM_SHARED((R, F), jnp.float32),
          pltpu.SemaphoreType.DMA,
          pltpu.SemaphoreType.DMA,
          pltpu.SemaphoreType.DMA,
      ],
  )


def _make_deg(total_rows):
  def body(edges, zer, out, didx, ones_v, acc, semi, sems):
    c = lax.axis_index("c")
    s = lax.axis_index("s")
    sl = pl.ds(s * ZROWS, ZROWS)
    for i in range(CHUNK // 16):
      ones_v[pl.ds(i * 16, 16)] = jnp.ones((16,), jnp.float32)
    pltpu.sync_copy(zer, acc.at[sl])
    plsc.subcore_barrier()
    rpt = total_rows // (NC * NS)
    base = (s * NC + c) * rpt

    @pl.loop(0, rpt // D)
    def _sup(i):
      g0 = base + i * D
      il = [pltpu.async_copy(edges.at[g0 + b, 1], didx.at[b], semi)
            for b in range(D)]
      sl_ = []
      for b in range(D):
        il[b].wait()
        sl_.append(pltpu.async_copy(ones_v, acc.at[didx.at[b]],
                                    sems, add=True))
      for b in range(D):
        sl_[b].wait()

    plsc.subcore_barrier()
    pl.when(c == 0)(lambda: pltpu.sync_copy(acc.at[sl], out.at[0, sl]))
    pl.when(c == 1)(lambda: pltpu.sync_copy(acc.at[sl], out.at[1, sl]))

  return pl.kernel(
      body,
      out_type=jax.ShapeDtypeStruct((NC, R), jnp.float32),
      mesh=_mesh,
      compiler_params=pltpu.CompilerParams(use_tc_tiling_on_sc=False),
      scratch_types=[
          pltpu.VMEM((D, CHUNK), jnp.int32),
          pltpu.VMEM((CHUNK,), jnp.float32),
          pltpu.VMEM_SHARED((R,), jnp.float32),
          pltpu.SemaphoreType.DMA,
          pltpu.SemaphoreType.DMA,
      ],
  )


def _mm1_body(x_ref, w_ref, o_ref):
  o_ref[...] = jnp.dot(x_ref[...], w_ref[...],
                       preferred_element_type=jnp.float32)


def _scale1_body(h_ref, dp0_ref, dp1_ref, hsa_ref, hsb_ref, dis_ref):
  dis = lax.rsqrt(dp0_ref[...] + dp1_ref[...] + 1.0)
  hs = h_ref[...] * dis
  hsa_ref[...] = hs[:, :16]
  hsb_ref[...] = hs[:, 16:]
  dis_ref[...] = dis


def _mid_body(a0_ref, a1_ref, hsa_ref, hsb_ref, dis_ref, w2_ref, b1_ref,
              hs2a_ref, hs2b_ref):
  dis = dis_ref[...]
  b1 = b1_ref[...]
  r0 = jnp.maximum((a0_ref[...] + hsa_ref[...]) * dis + b1[:, :16], 0.0)
  r1 = jnp.maximum((a1_ref[...] + hsb_ref[...]) * dis + b1[:, 16:], 0.0)
  w2 = w2_ref[...]
  h2 = (jnp.dot(r0, w2[:16, :], preferred_element_type=jnp.float32)
        + jnp.dot(r1, w2[16:, :], preferred_element_type=jnp.float32))
  hs2 = h2 * dis
  zpad = jnp.zeros((hs2.shape[0], 6), jnp.float32)
  hs2a_ref[...] = jnp.concatenate([hs2[:, :10], zpad], axis=-1)
  hs2b_ref[...] = jnp.concatenate([hs2[:, 10:], zpad], axis=-1)


def _post_body(a0_ref, a1_ref, hs2a_ref, hs2b_ref, dis_ref, b2_ref, o_ref):
  dis = dis_ref[...]
  b2 = b2_ref[...]
  v0 = (a0_ref[...] + hs2a_ref[...])[:, :10] * dis + b2[:, :10]
  v1 = (a1_ref[...] + hs2b_ref[...])[:, :10] * dis + b2[:, 10:]
  o_ref[...] = jnp.concatenate([v0, v1], axis=-1)


def _row_block(F):
  return pl.BlockSpec((BN, F), lambda i: (i, 0))


def _full_block(shape):
  return pl.BlockSpec(shape, lambda i: (0, 0))


def kernel(x, edge_index, W1, b1, W2, b2):
  x = x.astype(jnp.float32)
  ei = edge_index.astype(jnp.int32)
  E = ei.shape[1]
  group = NC * NS * SUP
  E_pad = ((E + group - 1) // group) * group
  pad = E_pad - E
  src = jnp.concatenate([ei[0], jnp.zeros((pad,), jnp.int32)])
  dst = jnp.concatenate([ei[1], jnp.full((pad,), N, jnp.int32)])
  edges = jnp.stack([src.reshape(-1, CHUNK), dst.reshape(-1, CHUNK)], axis=1)
  total_rows = E_pad // CHUNK
  z16 = jnp.zeros((ZROWS, 16), jnp.float32)
  zflat = jnp.zeros((ZROWS,), jnp.float32)

  grid = (N // BN,)

  degp = _make_deg(total_rows)(edges, zflat)
  h1 = pl.pallas_call(
      _mm1_body, grid=grid,
      in_specs=[_row_block(20), _full_block((20, 32))],
      out_specs=_row_block(32),
      out_shape=jax.ShapeDtypeStruct((N, 32), jnp.float32))(x, W1)

  dp0 = degp[0, :N].reshape(N, 1)
  dp1 = degp[1, :N].reshape(N, 1)
  hsa, hsb, dis = pl.pallas_call(
      _scale1_body, grid=grid,
      in_specs=[_row_block(32), _row_block(1), _row_block(1)],
      out_specs=[_row_block(16), _row_block(16), _row_block(1)],
      out_shape=[jax.ShapeDtypeStruct((N, 16), jnp.float32),
                 jax.ShapeDtypeStruct((N, 16), jnp.float32),
                 jax.ShapeDtypeStruct((N, 1), jnp.float32)])(h1, dp0, dp1)

  agg1 = _make_scatter(16, total_rows)(edges, hsa, hsb, z16)
  a10 = agg1[0, :N]
  a11 = agg1[1, :N]

  hs2a, hs2b = pl.pallas_call(
      _mid_body, grid=grid,
      in_specs=[_row_block(16), _row_block(16), _row_block(16),
                _row_block(16), _row_block(1), _full_block((32, 20)),
                _full_block((1, 32))],
      out_specs=[_row_block(16), _row_block(16)],
      out_shape=[jax.ShapeDtypeStruct((N, 16), jnp.float32),
                 jax.ShapeDtypeStruct((N, 16), jnp.float32)])(
          a10, a11, hsa, hsb, dis, W2, b1.reshape(1, 32))

  agg2 = _make_scatter(16, total_rows)(edges, hs2a, hs2b, z16)
  a20 = agg2[0, :N]
  a21 = agg2[1, :N]

  out = pl.pallas_call(
      _post_body, grid=grid,
      in_specs=[_row_block(16), _row_block(16), _row_block(16),
                _row_block(16), _row_block(1), _full_block((1, 20))],
      out_specs=_row_block(20),
      out_shape=jax.ShapeDtypeStruct((N, 20), jnp.float32))(
          a20, a21, hs2a, hs2b, dis, b2.reshape(1, 20))
  return out
